# baseline (speedup 1.0000x reference)
import functools

import jax
import jax.numpy as jnp
from jax import lax
from jax.experimental import pallas as pl
from jax.experimental.pallas import tpu as pltpu
from jax.experimental.pallas import tpu_sc as plsc

B = 16384
D = 64
DP = 128
OUT = 128
V = 1000000
PBLK = 1024
SUB = 48
TBLK = PBLK * SUB
NTB = (V + TBLK - 1) // TBLK
PR = NTB * (TBLK // 4)
NC = 2
NS = 16
NW = NC * NS
BPW = B // NW
CHUNK = 128
NCHUNK = BPW // CHUNK
BLK = 2048


def _pack_body(tt_ref, eye_ref, o_ref):
    blk = tt_ref[...]
    eye = eye_ref[...]
    dn = (((0,), (0,)), ((), ()))
    for s in range(SUB):
        sub = blk[:, s * PBLK : (s + 1) * PBLK]
        t = lax.dot_general(sub, eye, dn,
                            preferred_element_type=jnp.float32)
        bits = lax.bitcast_convert_type(t, jnp.int32)
        hi = (bits + 0x8000) >> 16
        QR = PBLK // 4
        r0 = s * QR
        pk_lo = (hi[:QR] & 0xFFFF) | (hi[QR : 2 * QR] << 16)
        pk_hi = (hi[2 * QR : 3 * QR] & 0xFFFF) | (hi[3 * QR :] << 16)
        o_ref[pl.ds(r0, QR), :D] = lax.bitcast_convert_type(pk_lo, jnp.float32)
        o_ref[pl.ds(r0, QR), D:] = lax.bitcast_convert_type(pk_hi, jnp.float32)


def _tc_pack(tableT, eye):
    return pl.pallas_call(
        _pack_body,
        grid=(NTB,),
        in_specs=[
            pl.BlockSpec((D, TBLK), lambda j: (0, j)),
            pl.BlockSpec((D, D), lambda j: (0, 0)),
        ],
        out_specs=pl.BlockSpec((TBLK // 4, DP), lambda j: (j, 0)),
        out_shape=jax.ShapeDtypeStruct((PR, DP), jnp.float32),
    )(tableT, eye)


def _sc_gather(idx2d, packed):
    mesh = plsc.VectorSubcoreMesh(core_axis_name="c", subcore_axis_name="s")

    @functools.partial(
        pl.kernel,
        mesh=mesh,
        out_type=jax.ShapeDtypeStruct((B, DP), jnp.float32),
        scratch_types=[
            pltpu.VMEM((NCHUNK, CHUNK), jnp.int32),
            pltpu.VMEM((BPW, DP), jnp.float32),
            pltpu.SemaphoreType.DMA,
        ],
    )
    def k(idx_hbm, table_hbm, out_hbm, idx_v, rows_v, sem):
        wid = lax.axis_index("s") * NC + lax.axis_index("c")
        pltpu.sync_copy(idx_hbm.at[pl.ds(wid * NCHUNK, NCHUNK)], idx_v)
        copies = []
        for j in range(NCHUNK):
            copies.append(
                pltpu.async_copy(
                    table_hbm.at[idx_v.at[j]],
                    rows_v.at[pl.ds(j * CHUNK, CHUNK)],
                    sem,
                )
            )
        for c in copies:
            c.wait()
        pltpu.sync_copy(rows_v, out_hbm.at[pl.ds(wid * BPW, BPW)])

    return k(idx2d, packed)


def _mm_body(e_ref, q_ref, w2_ref, b_ref, o_ref):
    bits = lax.bitcast_convert_type(e_ref[...], jnp.int32)
    e_lo = lax.bitcast_convert_type(bits << 16, jnp.float32)
    e_hi = lax.bitcast_convert_type(
        bits & jnp.int32(-65536), jnp.float32
    )
    q = q_ref[...].astype(jnp.int32)
    e_sel = jnp.where((q & 1) == 1, e_hi, e_lo)
    lane = lax.broadcasted_iota(jnp.int32, (BLK, DP), 1)
    keep = (lane >= D) == (q >= 2)
    e_m = jnp.where(keep, e_sel, 0.0)
    o_ref[...] = (
        lax.dot_general(
            e_m, w2_ref[...],
            (((1,), (0,)), ((), ())),
            preferred_element_type=jnp.float32,
        )
        + b_ref[...]
    )


def _tc_linear(e2, q, w2, fc_b2d):
    return pl.pallas_call(
        _mm_body,
        grid=(B // BLK,),
        in_specs=[
            pl.BlockSpec((BLK, DP), lambda i: (i, 0)),
            pl.BlockSpec((BLK, DP), lambda i: (i, 0)),
            pl.BlockSpec((DP, OUT), lambda i: (0, 0)),
            pl.BlockSpec((1, OUT), lambda i: (0, 0)),
        ],
        out_specs=pl.BlockSpec((BLK, OUT), lambda i: (i, 0)),
        out_shape=jax.ShapeDtypeStruct((B, OUT), jnp.float32),
    )(e2, q, w2, fc_b2d)


def kernel(_x, x, emb_table, fc_w, fc_b):
    xi = x.astype(jnp.int32)
    u_idx = ((xi >> 10) * (PBLK // 4) + (xi & (PBLK // 4 - 1))).reshape(
        NW * NCHUNK, CHUNK
    )
    q = jnp.broadcast_to(
        ((xi >> 8) & 3).astype(jnp.int8).reshape(B, 1), (B, DP)
    )
    eye = jnp.eye(D, dtype=jnp.float32)
    packed = _tc_pack(emb_table.T, eye)
    e2 = _sc_gather(u_idx, packed)
    w2 = jnp.concatenate([fc_w.T, fc_w.T], axis=0)
    return _tc_linear(e2, q, w2, fc_b.reshape(1, OUT))


# speedup vs baseline: 1.0140x; 1.0140x over previous
import functools

import jax
import jax.numpy as jnp
from jax import lax
from jax.experimental import pallas as pl
from jax.experimental.pallas import tpu as pltpu
from jax.experimental.pallas import tpu_sc as plsc

B = 16384
D = 64
DP = 128
OUT = 128
V = 1000000
PBLK = 1024
SUB = 32
TBLK = PBLK * SUB
NTB = (V + TBLK - 1) // TBLK
PR = NTB * (TBLK // 4)
NC = 2
NS = 16
NW = NC * NS
BPW = B // NW
CHUNK = 128
NCHUNK = BPW // CHUNK
BLK = 2048


def _pack_body(tt_ref, eye_ref, o_ref):
    blk = tt_ref[...]
    eye = eye_ref[...]
    dn = (((0,), (0,)), ((), ()))
    for s in range(SUB):
        sub = blk[:, s * PBLK : (s + 1) * PBLK]
        t = lax.dot_general(sub, eye, dn,
                            preferred_element_type=jnp.float32)
        bits = lax.bitcast_convert_type(t, jnp.int32)
        hi = (bits + 0x8000) >> 16
        QR = PBLK // 4
        r0 = s * QR
        pk_lo = (hi[:QR] & 0xFFFF) | (hi[QR : 2 * QR] << 16)
        pk_hi = (hi[2 * QR : 3 * QR] & 0xFFFF) | (hi[3 * QR :] << 16)
        o_ref[pl.ds(r0, QR), :D] = lax.bitcast_convert_type(pk_lo, jnp.float32)
        o_ref[pl.ds(r0, QR), D:] = lax.bitcast_convert_type(pk_hi, jnp.float32)


def _tc_pack(tableT, eye):
    return pl.pallas_call(
        _pack_body,
        grid=(NTB,),
        in_specs=[
            pl.BlockSpec((D, TBLK), lambda j: (0, j)),
            pl.BlockSpec((D, D), lambda j: (0, 0)),
        ],
        out_specs=pl.BlockSpec((TBLK // 4, DP), lambda j: (j, 0)),
        out_shape=jax.ShapeDtypeStruct((PR, DP), jnp.float32),
    )(tableT, eye)


def _sc_gather(idx2d, packed):
    mesh = plsc.VectorSubcoreMesh(core_axis_name="c", subcore_axis_name="s")

    @functools.partial(
        pl.kernel,
        mesh=mesh,
        out_type=jax.ShapeDtypeStruct((B, DP), jnp.float32),
        scratch_types=[
            pltpu.VMEM((NCHUNK, CHUNK), jnp.int32),
            pltpu.VMEM((BPW, DP), jnp.float32),
            pltpu.SemaphoreType.DMA,
        ],
    )
    def k(idx_hbm, table_hbm, out_hbm, idx_v, rows_v, sem):
        wid = lax.axis_index("s") * NC + lax.axis_index("c")
        pltpu.sync_copy(idx_hbm.at[pl.ds(wid * NCHUNK, NCHUNK)], idx_v)
        copies = []
        for j in range(NCHUNK):
            copies.append(
                pltpu.async_copy(
                    table_hbm.at[idx_v.at[j]],
                    rows_v.at[pl.ds(j * CHUNK, CHUNK)],
                    sem,
                )
            )
        for c in copies:
            c.wait()
        pltpu.sync_copy(rows_v, out_hbm.at[pl.ds(wid * BPW, BPW)])

    return k(idx2d, packed)


def _mm_body(e_ref, q_ref, w2_ref, b_ref, o_ref):
    bits = lax.bitcast_convert_type(e_ref[...], jnp.int32)
    e_lo = lax.bitcast_convert_type(bits << 16, jnp.float32)
    e_hi = lax.bitcast_convert_type(
        bits & jnp.int32(-65536), jnp.float32
    )
    q = q_ref[...].astype(jnp.int32)
    e_sel = jnp.where((q & 1) == 1, e_hi, e_lo)
    lane = lax.broadcasted_iota(jnp.int32, (BLK, DP), 1)
    keep = (lane >= D) == (q >= 2)
    e_m = jnp.where(keep, e_sel, 0.0)
    o_ref[...] = (
        lax.dot_general(
            e_m, w2_ref[...],
            (((1,), (0,)), ((), ())),
            preferred_element_type=jnp.float32,
        )
        + b_ref[...]
    )


def _tc_linear(e2, q, w2, fc_b2d):
    return pl.pallas_call(
        _mm_body,
        grid=(B // BLK,),
        in_specs=[
            pl.BlockSpec((BLK, DP), lambda i: (i, 0)),
            pl.BlockSpec((BLK, DP), lambda i: (i, 0)),
            pl.BlockSpec((DP, OUT), lambda i: (0, 0)),
            pl.BlockSpec((1, OUT), lambda i: (0, 0)),
        ],
        out_specs=pl.BlockSpec((BLK, OUT), lambda i: (i, 0)),
        out_shape=jax.ShapeDtypeStruct((B, OUT), jnp.float32),
    )(e2, q, w2, fc_b2d)


def kernel(_x, x, emb_table, fc_w, fc_b):
    xi = x.astype(jnp.int32)
    u_idx = ((xi >> 10) * (PBLK // 4) + (xi & (PBLK // 4 - 1))).reshape(
        NW * NCHUNK, CHUNK
    )
    q = jnp.broadcast_to(
        ((xi >> 8) & 3).astype(jnp.int8).reshape(B, 1), (B, DP)
    )
    eye = jnp.eye(D, dtype=jnp.float32)
    packed = _tc_pack(emb_table.T, eye)
    e2 = _sc_gather(u_idx, packed)
    w2 = jnp.concatenate([fc_w.T, fc_w.T], axis=0)
    return _tc_linear(e2, q, w2, fc_b.reshape(1, OUT))


# speedup vs baseline: 1.2536x; 1.2363x over previous
import functools

import jax
import jax.numpy as jnp
from jax import lax
from jax.experimental import pallas as pl
from jax.experimental.pallas import tpu as pltpu
from jax.experimental.pallas import tpu_sc as plsc

B = 16384
D = 64
DP = 128
OUT = 128
V = 1000000
PBLK = 1024
SUB = 32
TBLK = PBLK * SUB
NTB = (V + TBLK - 1) // TBLK
PR = NTB * (TBLK // 4)
NC = 2
NS = 16
NW = NC * NS
BPW = B // NW
CHUNK = 128
NCHUNK = BPW // CHUNK
BLK = 2048


def _pack_body(tt_ref, eye_ref, o_ref):
    blk = tt_ref[...]
    eye = eye_ref[...]
    dn = (((0,), (0,)), ((), ()))
    blk16 = blk.astype(jnp.bfloat16)
    eye16 = eye.astype(jnp.bfloat16)
    for s in range(SUB):
        sub = blk16[:, s * PBLK : (s + 1) * PBLK]
        t = lax.dot_general(sub, eye16, dn,
                            preferred_element_type=jnp.float32)
        bits = lax.bitcast_convert_type(t, jnp.int32)
        hi = bits >> 16
        QR = PBLK // 4
        r0 = s * QR
        pk_lo = (hi[:QR] & 0xFFFF) | (hi[QR : 2 * QR] << 16)
        pk_hi = (hi[2 * QR : 3 * QR] & 0xFFFF) | (hi[3 * QR :] << 16)
        o_ref[pl.ds(r0, QR), :D] = lax.bitcast_convert_type(pk_lo, jnp.float32)
        o_ref[pl.ds(r0, QR), D:] = lax.bitcast_convert_type(pk_hi, jnp.float32)


def _tc_pack(tableT, eye):
    return pl.pallas_call(
        _pack_body,
        grid=(NTB,),
        in_specs=[
            pl.BlockSpec((D, TBLK), lambda j: (0, j)),
            pl.BlockSpec((D, D), lambda j: (0, 0)),
        ],
        out_specs=pl.BlockSpec((TBLK // 4, DP), lambda j: (j, 0)),
        out_shape=jax.ShapeDtypeStruct((PR, DP), jnp.float32),
    )(tableT, eye)


def _sc_gather(idx2d, packed):
    mesh = plsc.VectorSubcoreMesh(core_axis_name="c", subcore_axis_name="s")

    @functools.partial(
        pl.kernel,
        mesh=mesh,
        out_type=jax.ShapeDtypeStruct((B, DP), jnp.float32),
        scratch_types=[
            pltpu.VMEM((NCHUNK, CHUNK), jnp.int32),
            pltpu.VMEM((BPW, DP), jnp.float32),
            pltpu.SemaphoreType.DMA,
        ],
    )
    def k(idx_hbm, table_hbm, out_hbm, idx_v, rows_v, sem):
        wid = lax.axis_index("s") * NC + lax.axis_index("c")
        pltpu.sync_copy(idx_hbm.at[pl.ds(wid * NCHUNK, NCHUNK)], idx_v)
        copies = []
        for j in range(NCHUNK):
            copies.append(
                pltpu.async_copy(
                    table_hbm.at[idx_v.at[j]],
                    rows_v.at[pl.ds(j * CHUNK, CHUNK)],
                    sem,
                )
            )
        for c in copies:
            c.wait()
        pltpu.sync_copy(rows_v, out_hbm.at[pl.ds(wid * BPW, BPW)])

    return k(idx2d, packed)


def _mm_body(e_ref, q_ref, w2_ref, b_ref, o_ref):
    bits = lax.bitcast_convert_type(e_ref[...], jnp.int32)
    e_lo = lax.bitcast_convert_type(bits << 16, jnp.float32)
    e_hi = lax.bitcast_convert_type(
        bits & jnp.int32(-65536), jnp.float32
    )
    q = q_ref[...].astype(jnp.int32)
    e_sel = jnp.where((q & 1) == 1, e_hi, e_lo)
    lane = lax.broadcasted_iota(jnp.int32, (BLK, DP), 1)
    keep = (lane >= D) == (q >= 2)
    e_m = jnp.where(keep, e_sel, 0.0)
    o_ref[...] = (
        lax.dot_general(
            e_m, w2_ref[...],
            (((1,), (0,)), ((), ())),
            preferred_element_type=jnp.float32,
        )
        + b_ref[...]
    )


def _tc_linear(e2, q, w2, fc_b2d):
    return pl.pallas_call(
        _mm_body,
        grid=(B // BLK,),
        in_specs=[
            pl.BlockSpec((BLK, DP), lambda i: (i, 0)),
            pl.BlockSpec((BLK, DP), lambda i: (i, 0)),
            pl.BlockSpec((DP, OUT), lambda i: (0, 0)),
            pl.BlockSpec((1, OUT), lambda i: (0, 0)),
        ],
        out_specs=pl.BlockSpec((BLK, OUT), lambda i: (i, 0)),
        out_shape=jax.ShapeDtypeStruct((B, OUT), jnp.float32),
    )(e2, q, w2, fc_b2d)


def kernel(_x, x, emb_table, fc_w, fc_b):
    xi = x.astype(jnp.int32)
    u_idx = ((xi >> 10) * (PBLK // 4) + (xi & (PBLK // 4 - 1))).reshape(
        NW * NCHUNK, CHUNK
    )
    q = jnp.broadcast_to(
        ((xi >> 8) & 3).astype(jnp.int8).reshape(B, 1), (B, DP)
    )
    eye = jnp.eye(D, dtype=jnp.float32)
    packed = _tc_pack(emb_table.T, eye)
    e2 = _sc_gather(u_idx, packed)
    w2 = jnp.concatenate([fc_w.T, fc_w.T], axis=0)
    return _tc_linear(e2, q, w2, fc_b.reshape(1, OUT))
